# baseline (speedup 1.0000x reference)
import functools

import jax
import jax.numpy as jnp
from jax import lax
from jax.experimental import pallas as pl
from jax.experimental.pallas import tpu as pltpu
from jax.experimental.pallas import tpu_sc as plsc

N = 10000
C = 128
NCORES = 2
NSUB = 16
NTILES = NCORES * NSUB
CHUNK = 128
NACC = 10112
ROWS_PER_TILE = NACC // NSUB
DEGW = 128

_mesh = plsc.VectorSubcoreMesh(core_axis_name="c", subcore_axis_name="s",
                               num_cores=NCORES, num_subcores=NSUB)


def _deg_body(nch, dst_hbm, zeros_hbm, ones_hbm, out_hbm, idx_v, ones_v, acc):
    c = lax.axis_index("c")
    s = lax.axis_index("s")
    wid = c * NSUB + s
    pltpu.sync_copy(dst_hbm.at[pl.ds(wid * nch, nch)], idx_v)
    pltpu.sync_copy(ones_hbm, ones_v)
    sl = pl.ds(s * ROWS_PER_TILE, ROWS_PER_TILE)
    pltpu.sync_copy(zeros_hbm.at[sl], acc.at[sl])
    plsc.subcore_barrier()

    def step(j, carry):
        pltpu.sync_copy(ones_v, acc.at[idx_v.at[j]], add=True)
        return carry

    lax.fori_loop(0, nch, step, 0)
    plsc.subcore_barrier()
    pltpu.sync_copy(acc.at[sl], out_hbm.at[pl.ds(c * NACC + s * ROWS_PER_TILE, ROWS_PER_TILE)])


SEG = 40


def _agg_body(n_a, n_b, g_hbm, src_hbm, dst_hbm, zeros_hbm, out_hbm,
              idx_s, idx_d, rows, sem_a, sem_b, acc):
    c = lax.axis_index("c")
    s = lax.axis_index("s")
    sl = pl.ds(s * ROWS_PER_TILE, ROWS_PER_TILE)
    pltpu.sync_copy(zeros_hbm.at[sl], acc.at[sl])
    plsc.subcore_barrier()

    base_ch = (1 - c) * (s * n_a) + c * (NSUB * n_a + s * n_b)
    nseg = (1 - c) * (n_a // SEG) + c * (n_b // SEG)

    def seg_body(g, carry):
        base = base_ch + g * SEG
        pltpu.sync_copy(src_hbm.at[pl.ds(base, SEG)], idx_s)
        pltpu.sync_copy(dst_hbm.at[pl.ds(base, SEG)], idx_d)

        def step(j, carry2):
            pltpu.async_copy(g_hbm.at[idx_s.at[j]], rows.at[0], sem_a).wait()
            pltpu.sync_copy(rows.at[0], acc.at[idx_d.at[j]], add=True)
            return carry2

        lax.fori_loop(0, SEG, step, 0)
        return carry

    lax.fori_loop(0, nseg, seg_body, 0)
    plsc.subcore_barrier()
    pltpu.sync_copy(acc.at[sl], out_hbm.at[pl.ds(c * NACC + s * ROWS_PER_TILE, ROWS_PER_TILE)])


def _k1_body(x_ref, w_ref, d0_ref, d1_ref, g_ref, dinv_ref):
    deg = d0_ref[:, 0:1] + d1_ref[:, 0:1] + 1.0
    dinv = lax.rsqrt(deg)
    h = jnp.dot(x_ref[...], w_ref[...], preferred_element_type=jnp.float32)
    g_ref[...] = h * dinv
    dinv_ref[...] = jnp.broadcast_to(dinv, g_ref.shape)


def _k2_body(p0_ref, p1_ref, g1_ref, dinv_ref, b_ref, w_ref, g2_ref):
    dinv = dinv_ref[...]
    z = dinv * (p0_ref[...] + p1_ref[...] + g1_ref[...]) + b_ref[...]
    z = jnp.maximum(z, 0.0)
    g2_ref[...] = jnp.dot(z, w_ref[...], preferred_element_type=jnp.float32) * dinv


def _k3_body(p0_ref, p1_ref, g2_ref, dinv_ref, b_ref, out_ref):
    out_ref[...] = dinv_ref[...] * (p0_ref[...] + p1_ref[...] + g2_ref[...]) + b_ref[...]


_BLK = 1000
_GRID = (N // _BLK,)


def _row_spec(w):
    return pl.BlockSpec((_BLK, w), lambda i: (i, 0))


def _full_spec(r, c):
    return pl.BlockSpec((r, c), lambda i: (0, 0))


def kernel(x, edge_index, W1, b1, W2, b2):
    E = edge_index.shape[1]
    nch = SEG * pl.cdiv(E, NTILES * CHUNK * SEG)
    tot = 2 * nch
    n_a = SEG * max(1, (3 * tot // 4) // SEG)
    n_b = tot - n_a
    totch = NSUB * tot
    epad = NSUB * tot * CHUNK - E
    ei = edge_index.astype(jnp.int32)
    src = jnp.concatenate([ei[0], jnp.zeros((epad,), jnp.int32)]).reshape(totch, CHUNK)
    dst = jnp.concatenate([ei[1], jnp.full((epad,), N, jnp.int32)]).reshape(totch, CHUNK)
    zeros16 = jnp.zeros((NACC, DEGW), jnp.float32)
    ones16 = jnp.ones((CHUNK, DEGW), jnp.float32)
    zerosC = jnp.zeros((NACC, C), jnp.float32)

    deg_k = pl.kernel(
        functools.partial(_deg_body, nch),
        out_type=jax.ShapeDtypeStruct((NCORES * NACC, DEGW), jnp.float32),
        mesh=_mesh,
        scratch_types=[
            pltpu.VMEM((nch, CHUNK), jnp.int32),
            pltpu.VMEM((CHUNK, DEGW), jnp.float32),
            pltpu.VMEM_SHARED((NACC, DEGW), jnp.float32),
        ],
    )
    deg2 = deg_k(dst, zeros16, ones16)
    d0 = deg2[0:N]
    d1 = deg2[NACC:NACC + N]

    agg_k = pl.kernel(
        functools.partial(_agg_body, n_a, n_b),
        out_type=jax.ShapeDtypeStruct((NCORES * NACC, C), jnp.float32),
        mesh=_mesh,
        scratch_types=[
            pltpu.VMEM((SEG, CHUNK), jnp.int32),
            pltpu.VMEM((SEG, CHUNK), jnp.int32),
            pltpu.VMEM((2, CHUNK, C), jnp.float32),
            pltpu.SemaphoreType.DMA,
            pltpu.SemaphoreType.DMA,
            pltpu.VMEM_SHARED((NACC, C), jnp.float32),
        ],
    )

    k1 = pl.pallas_call(
        _k1_body,
        grid=_GRID,
        in_specs=[_row_spec(C), _full_spec(C, C), _row_spec(DEGW), _row_spec(DEGW)],
        out_specs=[_row_spec(C), _row_spec(C)],
        out_shape=[jax.ShapeDtypeStruct((N, C), jnp.float32),
                   jax.ShapeDtypeStruct((N, C), jnp.float32)],
    )
    g1, dinvb = k1(x, W1, d0, d1)

    agg1 = agg_k(g1, src, dst, zerosC)
    p0 = agg1[0:N]
    p1 = agg1[NACC:NACC + N]

    k2 = pl.pallas_call(
        _k2_body,
        grid=_GRID,
        in_specs=[_row_spec(C), _row_spec(C), _row_spec(C), _row_spec(C),
                  _full_spec(1, C), _full_spec(C, C)],
        out_specs=_row_spec(C),
        out_shape=jax.ShapeDtypeStruct((N, C), jnp.float32),
    )
    g2 = k2(p0, p1, g1, dinvb, b1.reshape(1, C), W2)

    agg2 = agg_k(g2, src, dst, zerosC)
    q0 = agg2[0:N]
    q1 = agg2[NACC:NACC + N]

    k3 = pl.pallas_call(
        _k3_body,
        grid=_GRID,
        in_specs=[_row_spec(C), _row_spec(C), _row_spec(C), _row_spec(C),
                  _full_spec(1, C)],
        out_specs=_row_spec(C),
        out_shape=jax.ShapeDtypeStruct((N, C), jnp.float32),
    )
    return k3(q0, q1, g2, dinvb, b2.reshape(1, C))


# speedup vs baseline: 1.0151x; 1.0151x over previous
import functools

import jax
import jax.numpy as jnp
from jax import lax
from jax.experimental import pallas as pl
from jax.experimental.pallas import tpu as pltpu
from jax.experimental.pallas import tpu_sc as plsc

N = 10000
C = 128
NCORES = 2
NSUB = 16
NTILES = NCORES * NSUB
CHUNK = 128
NACC = 10112
ROWS_PER_TILE = NACC // NSUB
DEGW = 128

_mesh = plsc.VectorSubcoreMesh(core_axis_name="c", subcore_axis_name="s",
                               num_cores=NCORES, num_subcores=NSUB)


def _deg_body(nch, dst_hbm, zeros_hbm, ones_hbm, out_hbm, idx_v, ones_v, acc):
    c = lax.axis_index("c")
    s = lax.axis_index("s")
    wid = c * NSUB + s
    pltpu.sync_copy(dst_hbm.at[pl.ds(wid * nch, nch)], idx_v)
    pltpu.sync_copy(ones_hbm, ones_v)
    sl = pl.ds(s * ROWS_PER_TILE, ROWS_PER_TILE)
    pltpu.sync_copy(zeros_hbm.at[sl], acc.at[sl])
    plsc.subcore_barrier()

    def step(j, carry):
        pltpu.sync_copy(ones_v, acc.at[idx_v.at[j]], add=True)
        return carry

    lax.fori_loop(0, nch, step, 0)
    plsc.subcore_barrier()
    pltpu.sync_copy(acc.at[sl], out_hbm.at[pl.ds(c * NACC + s * ROWS_PER_TILE, ROWS_PER_TILE)])


SEG = 40


def _agg_body(n_a, n_b, g_hbm, src_hbm, dst_hbm, zeros_hbm, out_hbm,
              idx_s, idx_d, rows, sem_a, sem_b, acc):
    c = lax.axis_index("c")
    s = lax.axis_index("s")
    sl = pl.ds(s * ROWS_PER_TILE, ROWS_PER_TILE)
    pltpu.sync_copy(zeros_hbm.at[sl], acc.at[sl])
    plsc.subcore_barrier()

    base_ch = (1 - c) * (s * n_a) + c * (NSUB * n_a + s * n_b)
    nloc = n_a + (n_b - n_a) * c
    pltpu.sync_copy(src_hbm.at[pl.ds(base_ch, n_a)], idx_s)
    pltpu.sync_copy(dst_hbm.at[pl.ds(base_ch, n_a)], idx_d)

    def step(j, carry):
        pltpu.async_copy(g_hbm.at[idx_s.at[j]], rows.at[0], sem_a).wait()
        pltpu.sync_copy(rows.at[0], acc.at[idx_d.at[j]], add=True)
        return carry

    lax.fori_loop(0, nloc, step, 0)
    plsc.subcore_barrier()
    pltpu.sync_copy(acc.at[sl], out_hbm.at[pl.ds(c * NACC + s * ROWS_PER_TILE, ROWS_PER_TILE)])


def _k1_body(x_ref, w_ref, d0_ref, d1_ref, g_ref, dinv_ref):
    deg = d0_ref[:, 0:1] + d1_ref[:, 0:1] + 1.0
    dinv = lax.rsqrt(deg)
    h = jnp.dot(x_ref[...], w_ref[...], preferred_element_type=jnp.float32)
    g_ref[...] = h * dinv
    dinv_ref[...] = jnp.broadcast_to(dinv, g_ref.shape)


def _k2_body(p0_ref, p1_ref, g1_ref, dinv_ref, b_ref, w_ref, g2_ref):
    dinv = dinv_ref[...]
    z = dinv * (p0_ref[...] + p1_ref[...] + g1_ref[...]) + b_ref[...]
    z = jnp.maximum(z, 0.0)
    g2_ref[...] = jnp.dot(z, w_ref[...], preferred_element_type=jnp.float32) * dinv


def _k3_body(p0_ref, p1_ref, g2_ref, dinv_ref, b_ref, out_ref):
    out_ref[...] = dinv_ref[...] * (p0_ref[...] + p1_ref[...] + g2_ref[...]) + b_ref[...]


_BLK = 1000
_GRID = (N // _BLK,)


def _row_spec(w):
    return pl.BlockSpec((_BLK, w), lambda i: (i, 0))


def _full_spec(r, c):
    return pl.BlockSpec((r, c), lambda i: (0, 0))


def kernel(x, edge_index, W1, b1, W2, b2):
    E = edge_index.shape[1]
    nch = SEG * pl.cdiv(E, NTILES * CHUNK * SEG)
    tot = 2 * nch
    n_a = SEG * max(1, (3 * tot // 4) // SEG)
    n_b = tot - n_a
    totch = NSUB * tot + (n_a - n_b)
    epad = totch * CHUNK - E
    ei = edge_index.astype(jnp.int32)
    src = jnp.concatenate([ei[0], jnp.zeros((epad,), jnp.int32)]).reshape(totch, CHUNK)
    dst = jnp.concatenate([ei[1], jnp.full((epad,), N, jnp.int32)]).reshape(totch, CHUNK)
    zeros16 = jnp.zeros((NACC, DEGW), jnp.float32)
    ones16 = jnp.ones((CHUNK, DEGW), jnp.float32)
    zerosC = jnp.zeros((NACC, C), jnp.float32)

    deg_k = pl.kernel(
        functools.partial(_deg_body, nch),
        out_type=jax.ShapeDtypeStruct((NCORES * NACC, DEGW), jnp.float32),
        mesh=_mesh,
        scratch_types=[
            pltpu.VMEM((nch, CHUNK), jnp.int32),
            pltpu.VMEM((CHUNK, DEGW), jnp.float32),
            pltpu.VMEM_SHARED((NACC, DEGW), jnp.float32),
        ],
    )
    deg2 = deg_k(dst, zeros16, ones16)
    d0 = deg2[0:N]
    d1 = deg2[NACC:NACC + N]

    agg_k = pl.kernel(
        functools.partial(_agg_body, n_a, n_b),
        out_type=jax.ShapeDtypeStruct((NCORES * NACC, C), jnp.float32),
        mesh=_mesh,
        scratch_types=[
            pltpu.VMEM((n_a, CHUNK), jnp.int32),
            pltpu.VMEM((n_a, CHUNK), jnp.int32),
            pltpu.VMEM((1, CHUNK, C), jnp.float32),
            pltpu.SemaphoreType.DMA,
            pltpu.SemaphoreType.DMA,
            pltpu.VMEM_SHARED((NACC, C), jnp.float32),
        ],
    )

    k1 = pl.pallas_call(
        _k1_body,
        grid=_GRID,
        in_specs=[_row_spec(C), _full_spec(C, C), _row_spec(DEGW), _row_spec(DEGW)],
        out_specs=[_row_spec(C), _row_spec(C)],
        out_shape=[jax.ShapeDtypeStruct((N, C), jnp.float32),
                   jax.ShapeDtypeStruct((N, C), jnp.float32)],
    )
    g1, dinvb = k1(x, W1, d0, d1)

    agg1 = agg_k(g1, src, dst, zerosC)
    p0 = agg1[0:N]
    p1 = agg1[NACC:NACC + N]

    k2 = pl.pallas_call(
        _k2_body,
        grid=_GRID,
        in_specs=[_row_spec(C), _row_spec(C), _row_spec(C), _row_spec(C),
                  _full_spec(1, C), _full_spec(C, C)],
        out_specs=_row_spec(C),
        out_shape=jax.ShapeDtypeStruct((N, C), jnp.float32),
    )
    g2 = k2(p0, p1, g1, dinvb, b1.reshape(1, C), W2)

    agg2 = agg_k(g2, src, dst, zerosC)
    q0 = agg2[0:N]
    q1 = agg2[NACC:NACC + N]

    k3 = pl.pallas_call(
        _k3_body,
        grid=_GRID,
        in_specs=[_row_spec(C), _row_spec(C), _row_spec(C), _row_spec(C),
                  _full_spec(1, C)],
        out_specs=_row_spec(C),
        out_shape=jax.ShapeDtypeStruct((N, C), jnp.float32),
    )
    return k3(q0, q1, g2, dinvb, b2.reshape(1, C))


# speedup vs baseline: 1.0154x; 1.0002x over previous
import functools

import jax
import jax.numpy as jnp
from jax import lax
from jax.experimental import pallas as pl
from jax.experimental.pallas import tpu as pltpu
from jax.experimental.pallas import tpu_sc as plsc

N = 10000
C = 128
NCORES = 2
NSUB = 16
NTILES = NCORES * NSUB
CHUNK = 128
NACC = 10112
ROWS_PER_TILE = NACC // NSUB
DEGW = 128

_mesh = plsc.VectorSubcoreMesh(core_axis_name="c", subcore_axis_name="s",
                               num_cores=NCORES, num_subcores=NSUB)


def _deg_body(nch, dst_hbm, zeros_hbm, ones_hbm, out_hbm, idx_v, ones_v, acc):
    c = lax.axis_index("c")
    s = lax.axis_index("s")
    wid = c * NSUB + s
    pltpu.sync_copy(dst_hbm.at[pl.ds(wid * nch, nch)], idx_v)
    pltpu.sync_copy(ones_hbm, ones_v)
    sl = pl.ds(s * ROWS_PER_TILE, ROWS_PER_TILE)
    pltpu.sync_copy(zeros_hbm.at[sl], acc.at[sl])
    plsc.subcore_barrier()

    def step(j, carry):
        pltpu.sync_copy(ones_v, acc.at[idx_v.at[j]], add=True)
        return carry

    lax.fori_loop(0, nch, step, 0)
    plsc.subcore_barrier()
    pltpu.sync_copy(acc.at[sl], out_hbm.at[pl.ds(c * NACC + s * ROWS_PER_TILE, ROWS_PER_TILE)])


SEG = 40


def _agg_body(n_a, n_b, g_hbm, src_hbm, dst_hbm, zeros_hbm, out_hbm,
              idx_s, idx_d, rows, sem_a, sem_b, acc):
    c = lax.axis_index("c")
    s = lax.axis_index("s")
    sl = pl.ds(s * ROWS_PER_TILE, ROWS_PER_TILE)
    pltpu.sync_copy(zeros_hbm.at[sl], acc.at[sl])
    plsc.subcore_barrier()

    base_ch = (1 - c) * (s * n_a) + c * (NSUB * n_a + s * n_b)
    nloc = n_a + (n_b - n_a) * c
    pltpu.sync_copy(src_hbm.at[pl.ds(base_ch, n_a)], idx_s)
    pltpu.sync_copy(dst_hbm.at[pl.ds(base_ch, n_a)], idx_d)

    def step(j, carry):
        pltpu.async_copy(g_hbm.at[idx_s.at[j]], rows.at[0], sem_a).wait()
        pltpu.sync_copy(rows.at[0], acc.at[idx_d.at[j]], add=True)
        return carry

    lax.fori_loop(0, nloc, step, 0)
    plsc.subcore_barrier()
    pltpu.sync_copy(acc.at[sl], out_hbm.at[pl.ds(c * NACC + s * ROWS_PER_TILE, ROWS_PER_TILE)])


def _k1_body(x_ref, w_ref, d0_ref, d1_ref, g_ref, dinv_ref):
    deg = d0_ref[:, 0:1] + d1_ref[:, 0:1] + 1.0
    dinv = lax.rsqrt(deg)
    h = jnp.dot(x_ref[...], w_ref[...], preferred_element_type=jnp.float32)
    g_ref[...] = h * dinv
    dinv_ref[...] = jnp.broadcast_to(dinv, g_ref.shape)


def _k2_body(p0_ref, p1_ref, g1_ref, dinv_ref, b_ref, w_ref, g2_ref):
    dinv = dinv_ref[...]
    z = dinv * (p0_ref[...] + p1_ref[...] + g1_ref[...]) + b_ref[...]
    z = jnp.maximum(z, 0.0)
    g2_ref[...] = jnp.dot(z, w_ref[...], preferred_element_type=jnp.float32) * dinv


def _k3_body(p0_ref, p1_ref, g2_ref, dinv_ref, b_ref, out_ref):
    out_ref[...] = dinv_ref[...] * (p0_ref[...] + p1_ref[...] + g2_ref[...]) + b_ref[...]


_BLK = 1000
_GRID = (N // _BLK,)


def _row_spec(w):
    return pl.BlockSpec((_BLK, w), lambda i: (i, 0))


def _full_spec(r, c):
    return pl.BlockSpec((r, c), lambda i: (0, 0))


def kernel(x, edge_index, W1, b1, W2, b2):
    E = edge_index.shape[1]
    nch = SEG * pl.cdiv(E, NTILES * CHUNK * SEG)
    tot = 2 * nch
    n_a = SEG * max(1, (3 * tot // 4) // SEG)
    n_b = tot - n_a
    totch = NSUB * tot + (n_a - n_b)
    epad = totch * CHUNK - E
    ei = edge_index.astype(jnp.int32)
    src = jnp.concatenate([ei[0], jnp.zeros((epad,), jnp.int32)]).reshape(totch, CHUNK)
    pad_dst = N + jnp.arange(epad, dtype=jnp.int32) % (NACC - N)
    dst = jnp.concatenate([ei[1], pad_dst]).reshape(totch, CHUNK)
    zeros16 = jnp.zeros((NACC, DEGW), jnp.float32)
    ones16 = jnp.ones((CHUNK, DEGW), jnp.float32)
    zerosC = jnp.zeros((NACC, C), jnp.float32)

    deg_k = pl.kernel(
        functools.partial(_deg_body, nch),
        out_type=jax.ShapeDtypeStruct((NCORES * NACC, DEGW), jnp.float32),
        mesh=_mesh,
        scratch_types=[
            pltpu.VMEM((nch, CHUNK), jnp.int32),
            pltpu.VMEM((CHUNK, DEGW), jnp.float32),
            pltpu.VMEM_SHARED((NACC, DEGW), jnp.float32),
        ],
    )
    deg2 = deg_k(dst, zeros16, ones16)
    d0 = deg2[0:N]
    d1 = deg2[NACC:NACC + N]

    agg_k = pl.kernel(
        functools.partial(_agg_body, n_a, n_b),
        out_type=jax.ShapeDtypeStruct((NCORES * NACC, C), jnp.float32),
        mesh=_mesh,
        scratch_types=[
            pltpu.VMEM((n_a, CHUNK), jnp.int32),
            pltpu.VMEM((n_a, CHUNK), jnp.int32),
            pltpu.VMEM((1, CHUNK, C), jnp.float32),
            pltpu.SemaphoreType.DMA,
            pltpu.SemaphoreType.DMA,
            pltpu.VMEM_SHARED((NACC, C), jnp.float32),
        ],
    )

    k1 = pl.pallas_call(
        _k1_body,
        grid=_GRID,
        in_specs=[_row_spec(C), _full_spec(C, C), _row_spec(DEGW), _row_spec(DEGW)],
        out_specs=[_row_spec(C), _row_spec(C)],
        out_shape=[jax.ShapeDtypeStruct((N, C), jnp.float32),
                   jax.ShapeDtypeStruct((N, C), jnp.float32)],
    )
    g1, dinvb = k1(x, W1, d0, d1)

    agg1 = agg_k(g1, src, dst, zerosC)
    p0 = agg1[0:N]
    p1 = agg1[NACC:NACC + N]

    k2 = pl.pallas_call(
        _k2_body,
        grid=_GRID,
        in_specs=[_row_spec(C), _row_spec(C), _row_spec(C), _row_spec(C),
                  _full_spec(1, C), _full_spec(C, C)],
        out_specs=_row_spec(C),
        out_shape=jax.ShapeDtypeStruct((N, C), jnp.float32),
    )
    g2 = k2(p0, p1, g1, dinvb, b1.reshape(1, C), W2)

    agg2 = agg_k(g2, src, dst, zerosC)
    q0 = agg2[0:N]
    q1 = agg2[NACC:NACC + N]

    k3 = pl.pallas_call(
        _k3_body,
        grid=_GRID,
        in_specs=[_row_spec(C), _row_spec(C), _row_spec(C), _row_spec(C),
                  _full_spec(1, C)],
        out_specs=_row_spec(C),
        out_shape=jax.ShapeDtypeStruct((N, C), jnp.float32),
    )
    return k3(q0, q1, g2, dinvb, b2.reshape(1, C))


# speedup vs baseline: 1.4535x; 1.4315x over previous
import functools

import jax
import jax.numpy as jnp
from jax import lax
from jax.experimental import pallas as pl
from jax.experimental.pallas import tpu as pltpu
from jax.experimental.pallas import tpu_sc as plsc

N = 10000
C = 128
NCORES = 2
NSUB = 16
NTILES = NCORES * NSUB
CHUNK = 128
NACC = 10112
ROWS_PER_TILE = NACC // NSUB
DEGW = 128

_mesh = plsc.VectorSubcoreMesh(core_axis_name="c", subcore_axis_name="s",
                               num_cores=NCORES, num_subcores=NSUB)


def _deg_body(nch, dst_hbm, zeros_hbm, ones_hbm, out_hbm, idx_v, ones_v, acc):
    c = lax.axis_index("c")
    s = lax.axis_index("s")
    wid = c * NSUB + s
    pltpu.sync_copy(dst_hbm.at[wid], idx_v)
    pltpu.sync_copy(ones_hbm, ones_v)
    sl = pl.ds(s * ROWS_PER_TILE, ROWS_PER_TILE)
    pltpu.sync_copy(zeros_hbm.at[sl], acc.at[sl])
    plsc.subcore_barrier()

    def step(j, carry):
        pltpu.sync_copy(ones_v, acc.at[idx_v.at[j]], add=True)
        return carry

    lax.fori_loop(0, nch, step, 0)
    plsc.subcore_barrier()
    pltpu.sync_copy(acc.at[sl], out_hbm.at[pl.ds(c * NACC + s * ROWS_PER_TILE, ROWS_PER_TILE)])


def _agg_body(nch, g_hbm, src_hbm, dst_hbm, zeros_hbm, out_hbm,
              idx_s, idx_d, rows, sem, acc):
    c = lax.axis_index("c")
    s = lax.axis_index("s")
    wid = c * NSUB + s
    pltpu.sync_copy(src_hbm.at[wid], idx_s)
    pltpu.sync_copy(dst_hbm.at[wid], idx_d)
    sl = pl.ds(s * ROWS_PER_TILE, ROWS_PER_TILE)
    pltpu.sync_copy(zeros_hbm.at[sl], acc.at[sl])
    plsc.subcore_barrier()

    def step(j, carry):
        pltpu.async_copy(g_hbm.at[idx_s.at[j]], rows, sem).wait()
        pltpu.sync_copy(rows, acc.at[idx_d.at[j]], add=True)
        return carry

    lax.fori_loop(0, nch, step, 0)
    plsc.subcore_barrier()
    pltpu.sync_copy(acc.at[sl], out_hbm.at[pl.ds(c * NACC + s * ROWS_PER_TILE, ROWS_PER_TILE)])


def _k1_body(x_ref, w_ref, d0_ref, d1_ref, g_ref, dinv_ref):
    deg = d0_ref[:, 0:1] + d1_ref[:, 0:1] + 1.0
    dinv = lax.rsqrt(deg)
    h = jnp.dot(x_ref[...], w_ref[...], preferred_element_type=jnp.float32)
    g_ref[...] = h * dinv
    dinv_ref[...] = jnp.broadcast_to(dinv, g_ref.shape)


def _k2_body(p0_ref, p1_ref, g1_ref, dinv_ref, b_ref, w_ref, g2_ref):
    dinv = dinv_ref[...]
    z = dinv * (p0_ref[...] + p1_ref[...] + g1_ref[...]) + b_ref[...]
    z = jnp.maximum(z, 0.0)
    g2_ref[...] = jnp.dot(z, w_ref[...], preferred_element_type=jnp.float32) * dinv


def _k3_body(p0_ref, p1_ref, g2_ref, dinv_ref, b_ref, out_ref):
    out_ref[...] = dinv_ref[...] * (p0_ref[...] + p1_ref[...] + g2_ref[...]) + b_ref[...]


_BLK = 1000
_GRID = (N // _BLK,)


def _row_spec(w):
    return pl.BlockSpec((_BLK, w), lambda i: (i, 0))


def _full_spec(r, c):
    return pl.BlockSpec((r, c), lambda i: (0, 0))


def kernel(x, edge_index, W1, b1, W2, b2):
    E = edge_index.shape[1]
    nch = pl.cdiv(E, NTILES * CHUNK)
    epad = NTILES * nch * CHUNK - E
    ei = edge_index.astype(jnp.int32)
    src = jnp.concatenate([ei[0], jnp.zeros((epad,), jnp.int32)]).reshape(NTILES, nch, CHUNK)
    dst = jnp.concatenate([ei[1], jnp.full((epad,), N, jnp.int32)]).reshape(NTILES, nch, CHUNK)
    zeros16 = jnp.zeros((NACC, DEGW), jnp.float32)
    ones16 = jnp.ones((CHUNK, DEGW), jnp.float32)
    zerosC = jnp.zeros((NACC, C), jnp.float32)

    deg_k = pl.kernel(
        functools.partial(_deg_body, nch),
        out_type=jax.ShapeDtypeStruct((NCORES * NACC, DEGW), jnp.float32),
        mesh=_mesh,
        scratch_types=[
            pltpu.VMEM((nch, CHUNK), jnp.int32),
            pltpu.VMEM((CHUNK, DEGW), jnp.float32),
            pltpu.VMEM_SHARED((NACC, DEGW), jnp.float32),
        ],
    )
    deg2 = deg_k(dst, zeros16, ones16)
    d0 = deg2[0:N]
    d1 = deg2[NACC:NACC + N]

    agg_k = pl.kernel(
        functools.partial(_agg_body, nch),
        out_type=jax.ShapeDtypeStruct((NCORES * NACC, C), jnp.float32),
        mesh=_mesh,
        scratch_types=[
            pltpu.VMEM((nch, CHUNK), jnp.int32),
            pltpu.VMEM((nch, CHUNK), jnp.int32),
            pltpu.VMEM((CHUNK, C), jnp.float32),
            pltpu.SemaphoreType.DMA,
            pltpu.VMEM_SHARED((NACC, C), jnp.float32),
        ],
    )

    k1 = pl.pallas_call(
        _k1_body,
        grid=_GRID,
        in_specs=[_row_spec(C), _full_spec(C, C), _row_spec(DEGW), _row_spec(DEGW)],
        out_specs=[_row_spec(C), _row_spec(C)],
        out_shape=[jax.ShapeDtypeStruct((N, C), jnp.float32),
                   jax.ShapeDtypeStruct((N, C), jnp.float32)],
    )
    g1, dinvb = k1(x, W1, d0, d1)

    agg1 = agg_k(g1, src, dst, zerosC)
    p0 = agg1[0:N]
    p1 = agg1[NACC:NACC + N]

    k2 = pl.pallas_call(
        _k2_body,
        grid=_GRID,
        in_specs=[_row_spec(C), _row_spec(C), _row_spec(C), _row_spec(C),
                  _full_spec(1, C), _full_spec(C, C)],
        out_specs=_row_spec(C),
        out_shape=jax.ShapeDtypeStruct((N, C), jnp.float32),
    )
    g2 = k2(p0, p1, g1, dinvb, b1.reshape(1, C), W2)

    agg2 = agg_k(g2, src, dst, zerosC)
    q0 = agg2[0:N]
    q1 = agg2[NACC:NACC + N]

    k3 = pl.pallas_call(
        _k3_body,
        grid=_GRID,
        in_specs=[_row_spec(C), _row_spec(C), _row_spec(C), _row_spec(C),
                  _full_spec(1, C)],
        out_specs=_row_spec(C),
        out_shape=jax.ShapeDtypeStruct((N, C), jnp.float32),
    )
    return k3(q0, q1, g2, dinvb, b2.reshape(1, C))
